# baseline (speedup 1.0000x reference)
import numpy as np
import jax
import jax.numpy as jnp
from jax.experimental import pallas as pl
from jax.experimental.pallas import tpu as pltpu

_NF = 64
_NG = 32
_NB = 4
_NS = 6
_B = 4
_LVL = [_NB ** i for i in range(_NS + 1)]
_OFF = [int(v) for v in np.cumsum([0] + _LVL[:-1])]
_NN = sum(_LVL)

_BSZ = [8] + [_B * L for L in _LVL[1:]]
_BOFF = [int(v) for v in np.cumsum([0] + _BSZ[:-1])]
_RTOT = _BOFF[-1] + _BSZ[-1]


def _relu(x):
    return jnp.maximum(x, 0.0)


def _mm(x, W):
    return jnp.dot(x, W, preferred_element_type=jnp.float32)


def _bcast_rows(gg, l):
    if l == 0:
        return jnp.concatenate([gg, gg], axis=0)
    L = _LVL[l]
    return jnp.concatenate(
        [jnp.broadcast_to(gg[b:b + 1, :], (L, gg.shape[1])) for b in range(_B)],
        axis=0)


def _forward_kernel(rv_ref, *refs):
    refs = list(refs)
    agg_ref = refs.pop()
    rep_ref = refs.pop()
    out_ref = refs.pop()

    def take(n):
        nonlocal refs
        layers = []
        for _ in range(n):
            W = refs.pop(0)[...]
            b = refs.pop(0)[...]
            layers.append((W, b))
        return layers

    gpre = take(2)
    gpost = take(2)
    wproj = take(3)
    wmsg = take(3)
    wupd = take(3)

    pW1, pb1 = wproj[0]
    pW1x, pW1g = pW1[:_NF], pW1[_NF:]
    mW1, mb1 = wmsg[0]
    mW1x, mW1g = mW1[:_NF], mW1[_NF:]
    uW1, ub1 = wupd[0]
    uW1x, uW1a, uW1g = uW1[:_NF], uW1[_NF:2 * _NF], uW1[2 * _NF:]

    rv = rv_ref[...].reshape(_B, _NF)
    X = jnp.concatenate([rv, jnp.zeros((4, _NF), jnp.float32)], axis=0)

    agg_ref[pl.ds(0, 8), :] = jnp.zeros((8, _NF), jnp.float32)

    for k in range(1, _NS + 1):
        n_prev = _BOFF[k - 1] + _BSZ[k - 1]

        h = _relu(_mm(X, gpre[0][0]) + gpre[0][1])
        h = _relu(_mm(h, gpre[1][0]) + gpre[1][1])
        gsum = h[0:_B, :]
        for l in range(1, k):
            L = _LVL[l]
            o = _BOFF[l]
            gsum = gsum + jnp.concatenate(
                [jnp.sum(h[o + b * L:o + (b + 1) * L, :], axis=0,
                         keepdims=True) for b in range(_B)], axis=0)
        g = _relu(_mm(gsum, gpost[0][0]) + gpost[0][1])
        g = _relu(_mm(g, gpost[1][0]) + gpost[1][1])

        leaf0 = _BOFF[k - 1]
        n_leaf = _B * _LVL[k - 1]
        leaf = X[leaf0:leaf0 + n_leaf, :]
        gp = _mm(g, pW1g) + pb1
        p = _relu(_mm(leaf, pW1x) + _bcast_rows(gp, k - 1)[0:n_leaf, :])
        p = _relu(_mm(p, wproj[1][0]) + wproj[1][1])
        p = _relu(_mm(p, wproj[2][0]) + wproj[2][1])
        for c in range(_NB):
            rep_ref[pl.Slice(c, n_leaf, _NB), :] = p[:, c * _NF:(c + 1) * _NF]
        X = jnp.concatenate([X, rep_ref[pl.ds(0, n_leaf * _NB), :]], axis=0)

        gm = _mm(g, mW1g) + mb1
        Gm = jnp.concatenate([_bcast_rows(gm, l) for l in range(k)], axis=0)
        m = _relu(_mm(X[0:n_prev, :], mW1x) + Gm)
        m = _relu(_mm(m, wmsg[1][0]) + wmsg[1][1])
        M = _relu(_mm(m, wmsg[2][0]) + wmsg[2][1])

        a = jnp.zeros((_B, _NF), jnp.float32)
        for l in range(1, k + 1):
            src0 = _BOFF[l - 1]
            n_par = _B * _LVL[l - 1]
            t = a + M[src0:src0 + n_par, :]
            o = _BOFF[l]
            for c in range(_NB):
                agg_ref[pl.Slice(o + c, n_par, _NB), :] = t
            a = agg_ref[pl.ds(o, n_par * _NB), :]

        n_all = _BOFF[k] + _BSZ[k]
        A = agg_ref[pl.ds(0, n_all), :]
        gu = _mm(g, uW1g) + ub1
        Gu = jnp.concatenate([_bcast_rows(gu, l) for l in range(k + 1)], axis=0)
        u = _relu(_mm(X, uW1x) + _mm(A, uW1a) + Gu)
        u = _relu(_mm(u, wupd[1][0]) + wupd[1][1])
        X = _relu(_mm(u, wupd[2][0]) + wupd[2][1])

    for b in range(_B):
        for l in range(_NS + 1):
            L = _LVL[l]
            o = _BOFF[l] + (b if l == 0 else b * L)
            out_ref[pl.ds(b * _NN + _OFF[l], L), :] = X[o:o + L, :]


def kernel(random_vector, global_pre, global_post, proj, msg, update):
    flat = []
    for layers in (global_pre, global_post, proj, msg, update):
        for W, b in layers:
            flat.append(W)
            flat.append(b.reshape(1, -1))
    out = pl.pallas_call(
        _forward_kernel,
        out_shape=jax.ShapeDtypeStruct((_B * _NN, _NF), jnp.float32),
        scratch_shapes=[pltpu.VMEM((_B * _LVL[_NS], _NF), jnp.float32),
                        pltpu.VMEM((_RTOT, _NF), jnp.float32)],
    )(random_vector.reshape(_B, _NF), *flat)
    return out


# speedup vs baseline: 1.2121x; 1.2121x over previous
import numpy as np
import jax
import jax.numpy as jnp
from jax.experimental import pallas as pl
from jax.experimental.pallas import tpu as pltpu

_NF = 64
_NG = 32
_NB = 4
_NS = 6
_B = 4
_LVL = [_NB ** i for i in range(_NS + 1)]
_OFF = [int(v) for v in np.cumsum([0] + _LVL[:-1])]
_NN = sum(_LVL)

_BSZ = [8] + [_B * L for L in _LVL[1:]]
_BOFF = [int(v) for v in np.cumsum([0] + _BSZ[:-1])]
_RTOT = _BOFF[-1] + _BSZ[-1]


def _relu(x):
    return jnp.maximum(x, 0.0)


def _mm(x, W):
    return jnp.dot(x, W, preferred_element_type=jnp.float32)


def _bcast_rows(gg, l):
    if l == 0:
        return jnp.concatenate([gg, gg], axis=0)
    L = _LVL[l]
    return jnp.concatenate(
        [jnp.broadcast_to(gg[b:b + 1, :], (L, gg.shape[1])) for b in range(_B)],
        axis=0)


def _forward_kernel(rv_ref, *refs):
    refs = list(refs)
    agg_ref = refs.pop()
    rep_ref = refs.pop()
    out_ref = refs.pop()

    def take(n):
        nonlocal refs
        layers = []
        for _ in range(n):
            W = refs.pop(0)[...]
            b = refs.pop(0)[...]
            layers.append((W, b))
        return layers

    gpre = take(2)
    gpost = take(2)
    wproj = take(3)
    wmsg = take(3)
    wupd = take(3)

    rv = rv_ref[...].reshape(_B, _NF)
    X = jnp.concatenate([rv, jnp.zeros((4, _NF), jnp.float32)], axis=0)

    agg_ref[pl.ds(0, 8), :] = jnp.zeros((8, _NF), jnp.float32)

    for k in range(1, _NS + 1):
        n_prev = _BOFF[k - 1] + _BSZ[k - 1]

        h = _relu(_mm(X, gpre[0][0]) + gpre[0][1])
        h = _relu(_mm(h, gpre[1][0]) + gpre[1][1])
        gsum = h[0:_B, :]
        for l in range(1, k):
            L = _LVL[l]
            o = _BOFF[l]
            gsum = gsum + jnp.concatenate(
                [jnp.sum(h[o + b * L:o + (b + 1) * L, :], axis=0,
                         keepdims=True) for b in range(_B)], axis=0)
        g = _relu(_mm(gsum, gpost[0][0]) + gpost[0][1])
        g = _relu(_mm(g, gpost[1][0]) + gpost[1][1])

        leaf0 = _BOFF[k - 1]
        n_leaf = _B * _LVL[k - 1]
        leaf = X[leaf0:leaf0 + n_leaf, :]
        gleaf = _bcast_rows(g, k - 1)[0:n_leaf, :]
        p = _relu(_mm(jnp.concatenate([leaf, gleaf], axis=1), wproj[0][0])
                  + wproj[0][1])
        p = _relu(_mm(p, wproj[1][0]) + wproj[1][1])
        p = _relu(_mm(p, wproj[2][0]) + wproj[2][1])
        for c in range(_NB):
            rep_ref[pl.Slice(c, n_leaf, _NB), :] = p[:, c * _NF:(c + 1) * _NF]
        X = jnp.concatenate([X, rep_ref[pl.ds(0, n_leaf * _NB), :]], axis=0)

        Gm = jnp.concatenate([_bcast_rows(g, l) for l in range(k)], axis=0)
        m = _relu(_mm(jnp.concatenate([X[0:n_prev, :], Gm], axis=1),
                      wmsg[0][0]) + wmsg[0][1])
        m = _relu(_mm(m, wmsg[1][0]) + wmsg[1][1])
        M = _relu(_mm(m, wmsg[2][0]) + wmsg[2][1])

        a = jnp.zeros((_B, _NF), jnp.float32)
        for l in range(1, k + 1):
            src0 = _BOFF[l - 1]
            n_par = _B * _LVL[l - 1]
            t = a + M[src0:src0 + n_par, :]
            o = _BOFF[l]
            for c in range(_NB):
                agg_ref[pl.Slice(o + c, n_par, _NB), :] = t
            a = agg_ref[pl.ds(o, n_par * _NB), :]

        n_all = _BOFF[k] + _BSZ[k]
        A = agg_ref[pl.ds(0, n_all), :]
        Gu = jnp.concatenate([_bcast_rows(g, l) for l in range(k + 1)], axis=0)
        u = _relu(_mm(jnp.concatenate([X, A, Gu], axis=1), wupd[0][0])
                  + wupd[0][1])
        u = _relu(_mm(u, wupd[1][0]) + wupd[1][1])
        X = _relu(_mm(u, wupd[2][0]) + wupd[2][1])

    for b in range(_B):
        for l in range(_NS + 1):
            L = _LVL[l]
            o = _BOFF[l] + (b if l == 0 else b * L)
            out_ref[pl.ds(b * _NN + _OFF[l], L), :] = X[o:o + L, :]


def kernel(random_vector, global_pre, global_post, proj, msg, update):
    flat = []
    for layers in (global_pre, global_post, proj, msg, update):
        for W, b in layers:
            flat.append(W)
            flat.append(b.reshape(1, -1))
    out = pl.pallas_call(
        _forward_kernel,
        out_shape=jax.ShapeDtypeStruct((_B * _NN, _NF), jnp.float32),
        scratch_shapes=[pltpu.VMEM((_B * _LVL[_NS], _NF), jnp.float32),
                        pltpu.VMEM((_RTOT, _NF), jnp.float32)],
    )(random_vector.reshape(_B, _NF), *flat)
    return out
